# baseline (device time: 15460 ns/iter reference)
import jax
import jax.numpy as jnp
from jax import lax
from jax.experimental import pallas as pl
from jax.experimental.pallas import tpu as pltpu

N_DEV = 8
CH = 4


def kernel(x, k):
    b, s, c = x.shape
    taps = k.shape[0]
    halo = taps - 1
    rows = s // CH

    def body(x_ref, k_ref, out_ref, xv_ref, ov_ref, halo_ref,
             in_sems, out_sems, fix_sem, send_sem, recv_sem):
        my_i = lax.axis_index("i")

        credit_sem = pltpu.get_barrier_semaphore()

        cps = []
        for ci in range(CH):
            cp = pltpu.make_async_copy(
                x_ref.at[:, pl.ds(ci * rows, rows), :],
                xv_ref.at[:, pl.ds(ci * rows, rows), :],
                in_sems.at[ci],
            )
            cp.start()
            cps.append(cp)

        @pl.when(my_i > 0)
        def _():
            pl.semaphore_signal(
                credit_sem,
                inc=1,
                device_id=(my_i - 1,),
                device_id_type=pl.DeviceIdType.MESH,
            )

        send = pltpu.make_async_remote_copy(
            src_ref=x_ref.at[:, pl.ds(s - halo, halo), :],
            dst_ref=halo_ref,
            send_sem=send_sem,
            recv_sem=recv_sem,
            device_id=(lax.rem(my_i + 1, N_DEV),),
            device_id_type=pl.DeviceIdType.MESH,
        )

        @pl.when(my_i < N_DEV - 1)
        def _():
            pl.semaphore_wait(credit_sem, 1)
            send.start()

        kv = k_ref[...].astype(jnp.bfloat16)

        out_cps = []
        for ci in range(CH):
            cps[ci].wait()
            if ci == 0:
                xv = xv_ref[:, pl.ds(0, rows), :].astype(jnp.bfloat16)
                pad = jnp.concatenate(
                    [jnp.zeros((b, halo, c), jnp.bfloat16), xv], axis=1
                )
            else:
                pad = xv_ref[
                    :, pl.ds(ci * rows - halo, rows + halo), :
                ].astype(jnp.bfloat16)
            acc = pad[:, 0:rows, :] * kv[0]
            for t in range(1, taps):
                acc = acc + pad[:, t:t + rows, :] * kv[t]
            ov_ref[:, pl.ds(ci * rows, rows), :] = (
                acc * (1.0 / (1.0 + jnp.exp(-acc)))
            )
            ocp = pltpu.make_async_copy(
                ov_ref.at[:, pl.ds(ci * rows, rows), :],
                out_ref.at[:, pl.ds(ci * rows, rows), :],
                out_sems.at[ci],
            )
            ocp.start()
            out_cps.append(ocp)

        @pl.when(my_i > 0)
        def _():
            send.wait_recv()
            out_cps[0].wait()
            hv = halo_ref[...].astype(jnp.bfloat16)
            xv3 = xv_ref[:, pl.ds(0, halo), :].astype(jnp.bfloat16)
            pad3 = jnp.concatenate([hv, xv3], axis=1)
            accs = []
            for i in range(halo):
                a = pad3[:, i, :] * kv[0]
                for t in range(1, taps):
                    a = a + pad3[:, i + t, :] * kv[t]
                accs.append(a[:, None, :])
            a3 = jnp.concatenate(accs, axis=1)
            ov_ref[:, pl.ds(0, halo), :] = (
                a3 * (1.0 / (1.0 + jnp.exp(-a3)))
            )
            fcp = pltpu.make_async_copy(
                ov_ref.at[:, pl.ds(0, 8), :],
                out_ref.at[:, pl.ds(0, 8), :],
                fix_sem,
            )
            fcp.start()
            fcp.wait()

        @pl.when(my_i == 0)
        def _():
            out_cps[0].wait()
        for ci in range(1, CH):
            out_cps[ci].wait()

        @pl.when(my_i < N_DEV - 1)
        def _():
            send.wait_send()

    return pl.pallas_call(
        body,
        out_shape=jax.ShapeDtypeStruct((b, s, c), jnp.bfloat16),
        in_specs=[
            pl.BlockSpec(memory_space=pl.ANY),
            pl.BlockSpec(memory_space=pltpu.VMEM),
        ],
        out_specs=pl.BlockSpec(memory_space=pl.ANY),
        scratch_shapes=[
            pltpu.VMEM((b, s, c), x.dtype),
            pltpu.VMEM((b, s, c), jnp.bfloat16),
            pltpu.VMEM((b, halo, c), x.dtype),
            pltpu.SemaphoreType.DMA((CH,)),
            pltpu.SemaphoreType.DMA((CH,)),
            pltpu.SemaphoreType.DMA,
            pltpu.SemaphoreType.DMA,
            pltpu.SemaphoreType.DMA,
        ],
        compiler_params=pltpu.CompilerParams(collective_id=0),
    )(x, k)


# device time: 15423 ns/iter; 1.0024x vs baseline; 1.0024x over previous
import jax
import jax.numpy as jnp
from jax import lax
from jax.experimental import pallas as pl
from jax.experimental.pallas import tpu as pltpu

N_DEV = 8
CH = 4


def kernel(x, k):
    b, s, c = x.shape
    taps = k.shape[0]
    halo = taps - 1
    rows = s // CH

    def body(x_ref, k_ref, out_ref, xv_ref, ov_ref, halo_ref,
             in_sems, out_sems, fix_sem, send_sem, recv_sem):
        my_i = lax.axis_index("i")

        credit_sem = pltpu.get_barrier_semaphore()

        cps = []
        for ci in range(CH):
            cp = pltpu.make_async_copy(
                x_ref.at[:, pl.ds(ci * rows, rows), :],
                xv_ref.at[:, pl.ds(ci * rows, rows), :],
                in_sems.at[ci],
            )
            cp.start()
            cps.append(cp)

        @pl.when(my_i > 0)
        def _():
            pl.semaphore_signal(
                credit_sem,
                inc=1,
                device_id=(my_i - 1,),
                device_id_type=pl.DeviceIdType.MESH,
            )

        send = pltpu.make_async_remote_copy(
            src_ref=x_ref.at[:, pl.ds(s - halo, halo), :],
            dst_ref=halo_ref,
            send_sem=send_sem,
            recv_sem=recv_sem,
            device_id=(lax.rem(my_i + 1, N_DEV),),
            device_id_type=pl.DeviceIdType.MESH,
        )

        @pl.when(my_i < N_DEV - 1)
        def _():
            pl.semaphore_wait(credit_sem, 1)
            send.start()

        kv = k_ref[...].astype(jnp.bfloat16)

        out_cps = []
        for ci in range(CH):
            cps[ci].wait()
            if ci == 0:
                xv = xv_ref[:, pl.ds(0, rows), :].astype(jnp.bfloat16)
                pad = jnp.concatenate(
                    [jnp.zeros((b, halo, c), jnp.bfloat16), xv], axis=1
                )
            else:
                pad = xv_ref[
                    :, pl.ds(ci * rows - halo, rows + halo), :
                ].astype(jnp.bfloat16)
            acc = pad[:, 0:rows, :] * kv[0]
            for t in range(1, taps):
                acc = acc + pad[:, t:t + rows, :] * kv[t]
            ov_ref[:, pl.ds(ci * rows, rows), :] = (
                acc * (1.0 / (1.0 + jnp.exp(-acc)))
            )
            ocp = pltpu.make_async_copy(
                ov_ref.at[:, pl.ds(ci * rows, rows), :],
                out_ref.at[:, pl.ds(ci * rows, rows), :],
                out_sems.at[ci],
            )
            ocp.start()
            out_cps.append(ocp)

        @pl.when(my_i > 0)
        def _():
            send.wait_recv()
            out_cps[0].wait()
            hv = halo_ref[...].astype(jnp.bfloat16)
            xv3 = xv_ref[:, pl.ds(0, halo), :].astype(jnp.bfloat16)
            pad3 = jnp.concatenate([hv, xv3], axis=1)
            accs = []
            for i in range(halo):
                a = pad3[:, i, :] * kv[0]
                for t in range(1, taps):
                    a = a + pad3[:, i + t, :] * kv[t]
                accs.append(a[:, None, :])
            a3 = jnp.concatenate(accs, axis=1)
            ov_ref[:, pl.ds(0, halo), :] = (
                a3 * (1.0 / (1.0 + jnp.exp(-a3)))
            )
            fcp = pltpu.make_async_copy(
                ov_ref.at[:, pl.ds(0, 8), :],
                out_ref.at[:, pl.ds(0, 8), :],
                fix_sem,
            )
            fcp.start()
            fcp.wait()

        @pl.when(my_i == 0)
        def _():
            out_cps[0].wait()
        for ci in range(1, CH):
            out_cps[ci].wait()

        @pl.when(my_i < N_DEV - 1)
        def _():
            send.wait_send()

    return pl.pallas_call(
        body,
        out_shape=jax.ShapeDtypeStruct((b, s, c), jnp.bfloat16),
        in_specs=[
            pl.BlockSpec(memory_space=pltpu.MemorySpace.HBM),
            pl.BlockSpec(memory_space=pltpu.VMEM),
        ],
        out_specs=pl.BlockSpec(memory_space=pltpu.MemorySpace.HBM),
        scratch_shapes=[
            pltpu.VMEM((b, s, c), x.dtype),
            pltpu.VMEM((b, s, c), jnp.bfloat16),
            pltpu.VMEM((b, halo, c), x.dtype),
            pltpu.SemaphoreType.DMA((CH,)),
            pltpu.SemaphoreType.DMA((CH,)),
            pltpu.SemaphoreType.DMA,
            pltpu.SemaphoreType.DMA,
            pltpu.SemaphoreType.DMA,
        ],
        compiler_params=pltpu.CompilerParams(collective_id=0),
    )(x, k)


# device time: 13018 ns/iter; 1.1876x vs baseline; 1.1847x over previous
import jax
import jax.numpy as jnp
from jax import lax
from jax.experimental import pallas as pl
from jax.experimental.pallas import tpu as pltpu

N_DEV = 8
ROWS_A = 256


def kernel(x, k):
    b, s, c = x.shape
    taps = k.shape[0]
    halo = taps - 1

    def body(x_ref, k_ref, out_ref, halo_ref, send_sem, recv_sem):
        my_i = lax.axis_index("i")

        credit_sem = pltpu.get_barrier_semaphore()

        @pl.when(my_i > 0)
        def _():
            pl.semaphore_signal(
                credit_sem,
                inc=1,
                device_id=(my_i - 1,),
                device_id_type=pl.DeviceIdType.MESH,
            )

        kv = k_ref[...].astype(jnp.bfloat16)

        def conv(pad, n):
            acc = pad[:, 0:n, :] * kv[0]
            for t in range(1, taps):
                acc = acc + pad[:, t:t + n, :] * kv[t]
            return acc

        silu = lambda a: a * (1.0 / (1.0 + jnp.exp(-a)))

        xa = x_ref[:, pl.ds(0, ROWS_A), :].astype(jnp.bfloat16)
        pad_a = jnp.concatenate(
            [jnp.zeros((b, halo, c), jnp.bfloat16), xa], axis=1
        )
        acc_a = conv(pad_a, ROWS_A)
        out_ref[:, pl.ds(0, ROWS_A), :] = silu(acc_a)

        send = pltpu.make_async_remote_copy(
            src_ref=x_ref.at[:, pl.ds(s - halo, halo), :],
            dst_ref=halo_ref,
            send_sem=send_sem,
            recv_sem=recv_sem,
            device_id=(lax.rem(my_i + 1, N_DEV),),
            device_id_type=pl.DeviceIdType.MESH,
        )

        @pl.when(my_i < N_DEV - 1)
        def _():
            pl.semaphore_wait(credit_sem, 1)
            send.start()

        pad_b = x_ref[
            :, pl.ds(ROWS_A - halo, s - ROWS_A + halo), :
        ].astype(jnp.bfloat16)
        acc_b = conv(pad_b, s - ROWS_A)
        out_ref[:, pl.ds(ROWS_A, s - ROWS_A), :] = silu(acc_b)

        @pl.when(my_i > 0)
        def _():
            send.wait_recv()
            hv = halo_ref[...].astype(jnp.bfloat16)
            rows = []
            for i in range(halo):
                m = kv[0] * hv[:, i, :]
                for t in range(1, halo - i):
                    m = m + kv[t] * hv[:, i + t, :]
                rows.append(m[:, None, :])
            missing = jnp.concatenate(rows, axis=1)
            a3 = acc_a[:, 0:halo, :] + missing
            out_ref[:, pl.ds(0, halo), :] = silu(a3)

        @pl.when(my_i < N_DEV - 1)
        def _():
            send.wait_send()

    return pl.pallas_call(
        body,
        out_shape=jax.ShapeDtypeStruct((b, s, c), jnp.bfloat16),
        in_specs=[
            pl.BlockSpec(memory_space=pltpu.VMEM),
            pl.BlockSpec(memory_space=pltpu.VMEM),
        ],
        out_specs=pl.BlockSpec(memory_space=pltpu.VMEM),
        scratch_shapes=[
            pltpu.VMEM((b, halo, c), x.dtype),
            pltpu.SemaphoreType.DMA,
            pltpu.SemaphoreType.DMA,
        ],
        compiler_params=pltpu.CompilerParams(collective_id=0),
    )(x, k)
